# baseline (speedup 1.0000x reference)
import functools

import jax
import jax.numpy as jnp
from jax import lax
from jax.experimental import pallas as pl
from jax.experimental.pallas import tpu as pltpu
from jax.experimental.pallas import tpu_sc as plsc

T = 2048
H = 768
E = 64
S_EXP = 2
FFN_S = H * 3
FFN_R = H * 2
R = 64
TOPK = 2
NPAIR = T * TOPK

TT = 256
NT = T // TT
TM = 128
ALIGN = 8
XS_ROWS = 4672


def _kernel_a0(x_ref, rd_ref, ru_ref,
               rw_ref, pos_ref, offs_ref, cnts_ref, loss_ref,
               sel_scr, rank_scr, rw_scr, cnt_scr):
    i = pl.program_id(0)
    ids = lax.broadcasted_iota(jnp.int32, (TT, E), 1)

    @pl.when(i == 0)
    def _():
        cnt_scr[...] = jnp.zeros_like(cnt_scr)

    @pl.when(i < NT)
    def _phase0():
        xf = x_ref[...]
        lg = jnp.dot(jnp.dot(xf, rd_ref[...],
                             preferred_element_type=jnp.float32),
                     ru_ref[...], preferred_element_type=jnp.float32)
        v0 = jnp.max(lg, axis=1, keepdims=True)
        e0 = jnp.min(jnp.where(lg == v0, ids, E), axis=1, keepdims=True)
        lg2 = jnp.where(ids == e0, -jnp.inf, lg)
        v1 = jnp.max(lg2, axis=1, keepdims=True)
        e1 = jnp.min(jnp.where(lg2 == v1, ids, E), axis=1, keepdims=True)
        ed = jnp.exp(v1 - v0)
        denom = 1.0 + ed
        sl = pl.ds(i * TT, TT)
        sel_scr[sl, :] = jnp.concatenate([e0, e1], axis=1)
        rw_scr[sl, :] = jnp.concatenate([1.0 / denom, ed / denom], axis=1)

        c0 = (ids == e0).astype(jnp.float32)
        c1 = (ids == e1).astype(jnp.float32)
        m = c0 + c1
        lr = lax.broadcasted_iota(jnp.int32, (TT, TT), 0)
        lc = lax.broadcasted_iota(jnp.int32, (TT, TT), 1)
        ltri = (lr > lc).astype(jnp.float32)
        excl = jnp.dot(ltri, m,
                       preferred_element_type=jnp.float32) + cnt_scr[...]
        rank0 = jnp.sum(excl * c0, axis=1, keepdims=True)
        rank1 = jnp.sum(excl * c1, axis=1, keepdims=True)
        rank_scr[sl, :] = jnp.concatenate([rank0, rank1], axis=1)
        cnt_scr[...] = cnt_scr[...] + jnp.sum(m, axis=0, keepdims=True)

    @pl.when(i >= NT)
    def _phase1():
        cnt = cnt_scr[...]
        er = lax.broadcasted_iota(jnp.int32, (E, E), 0)
        ec = lax.broadcasted_iota(jnp.int32, (E, E), 1)
        utri = (er < ec).astype(jnp.float32)
        pc = jnp.ceil(cnt * (1.0 / ALIGN)) * float(ALIGN)
        offs = jnp.dot(pc, utri, preferred_element_type=jnp.float32)
        offs_ref[...] = offs.astype(jnp.int32)
        cnts_ref[...] = cnt.astype(jnp.int32)
        mean = NPAIR / E
        loss_ref[...] = (jnp.sum((cnt - mean) ** 2, keepdims=True)
                         .reshape(1, 1) / (E - 1))

        sl = pl.ds((i - NT) * TT, TT)
        sel = sel_scr[sl, :]
        rank = rank_scr[sl, :]
        on0 = (ids == sel[:, 0:1]).astype(jnp.float32)
        on1 = (ids == sel[:, 1:2]).astype(jnp.float32)
        pos0 = jnp.sum(on0 * offs, axis=1, keepdims=True) + rank[:, 0:1]
        pos1 = jnp.sum(on1 * offs, axis=1, keepdims=True) + rank[:, 1:2]
        pos_ref[...] = jnp.concatenate([pos0, pos1], axis=1).astype(jnp.int32)
        rw_ref[...] = rw_scr[sl, :]


def _run_kernel_a0(xf, rd, ru, interpret=False):
    def xmap(i):
        return (lax.rem(i, NT), 0)

    def omap(i):
        return (jnp.maximum(i - NT, 0), 0)

    return pl.pallas_call(
        _kernel_a0,
        grid=(2 * NT,),
        in_specs=[
            pl.BlockSpec((TT, H), xmap),
            pl.BlockSpec((H, R), lambda i: (0, 0)),
            pl.BlockSpec((R, E), lambda i: (0, 0)),
        ],
        out_specs=[
            pl.BlockSpec((TT, TOPK), omap),
            pl.BlockSpec((TT, TOPK), omap),
            pl.BlockSpec((1, E), lambda i: (0, 0)),
            pl.BlockSpec((1, E), lambda i: (0, 0)),
            pl.BlockSpec((1, 1), lambda i: (0, 0)),
        ],
        out_shape=[
            jax.ShapeDtypeStruct((T, TOPK), jnp.float32),
            jax.ShapeDtypeStruct((T, TOPK), jnp.int32),
            jax.ShapeDtypeStruct((1, E), jnp.int32),
            jax.ShapeDtypeStruct((1, E), jnp.int32),
            jax.ShapeDtypeStruct((1, 1), jnp.float32),
        ],
        scratch_shapes=[
            pltpu.VMEM((T, TOPK), jnp.int32),
            pltpu.VMEM((T, TOPK), jnp.float32),
            pltpu.VMEM((T, TOPK), jnp.float32),
            pltpu.VMEM((1, E), jnp.float32),
        ],
        interpret=interpret,
    )(xf, rd, ru)


def _kernel_a1(x_ref, norm_ref, w1_ref, w2_ref, w3_ref, shared_ref):
    xf = x_ref[...]
    inv = lax.rsqrt(jnp.mean(xf * xf, axis=1, keepdims=True) + 1e-6)
    acc = 2.0 * xf
    for s in range(S_EXP):
        hn = xf * inv * norm_ref[s:s + 1, :]
        g = jnp.dot(hn, w1_ref[s], preferred_element_type=jnp.float32)
        g = g * jax.nn.sigmoid(g)
        v = jnp.dot(hn, w3_ref[s], preferred_element_type=jnp.float32)
        acc = acc + jnp.dot(g * v, w2_ref[s],
                            preferred_element_type=jnp.float32)
    shared_ref[...] = acc * (1.0 / S_EXP)


def _run_kernel_a1(xf, norm_w, w1, w2, w3, interpret=False):
    return pl.pallas_call(
        _kernel_a1,
        grid=(NT,),
        in_specs=[
            pl.BlockSpec((TT, H), lambda i: (i, 0)),
            pl.BlockSpec((S_EXP, H), lambda i: (0, 0)),
            pl.BlockSpec((S_EXP, H, FFN_S), lambda i: (0, 0, 0)),
            pl.BlockSpec((S_EXP, FFN_S, H), lambda i: (0, 0, 0)),
            pl.BlockSpec((S_EXP, H, FFN_S), lambda i: (0, 0, 0)),
        ],
        out_specs=pl.BlockSpec((TT, H), lambda i: (i, 0)),
        out_shape=jax.ShapeDtypeStruct((T, H), jnp.float32),
        interpret=interpret,
    )(xf, norm_w, w1, w2, w3)


NW = 32
TPW = T // NW


def _sc_dispatch(x_hbm, pos0_hbm, pos1_hbm, xs_hbm, p0_v, p1_v, rows_v,
                 sem0, sem1):
    wid = lax.axis_index("s") * 2 + lax.axis_index("c")
    base = wid * TPW
    pltpu.sync_copy(pos0_hbm.at[pl.ds(base, TPW)], p0_v)
    pltpu.sync_copy(pos1_hbm.at[pl.ds(base, TPW)], p1_v)
    pltpu.sync_copy(x_hbm.at[pl.ds(base, TPW)], rows_v)
    c0 = pltpu.async_copy(rows_v, xs_hbm.at[p0_v], sem0)
    c1 = pltpu.async_copy(rows_v, xs_hbm.at[p1_v], sem1)
    c0.wait()
    c1.wait()


def _run_sc_dispatch(xf, pos0, pos1):
    mesh = plsc.VectorSubcoreMesh(core_axis_name="c", subcore_axis_name="s")
    k = functools.partial(
        pl.kernel, mesh=mesh,
        out_type=jax.ShapeDtypeStruct((XS_ROWS, H), jnp.float32),
        scratch_types=[
            pltpu.VMEM((TPW,), jnp.int32),
            pltpu.VMEM((TPW,), jnp.int32),
            pltpu.VMEM((TPW, H), jnp.float32),
            pltpu.SemaphoreType.DMA,
            pltpu.SemaphoreType.DMA,
        ],
    )(_sc_dispatch)
    return k(xf, pos0, pos1)


def _dispatch_jnp(xf, pos):
    tok = jnp.arange(NPAIR, dtype=jnp.int32) // TOPK
    return jnp.zeros((XS_ROWS, H), jnp.float32).at[pos.reshape(-1)].set(xf[tok])


def _kernel_b(offs_ref, cnts_ref, pos_ref, rwts_ref, xs_ref,
              w1_ref, w2_ref, w3_ref, shared_ref, ys_ref, out_ref):
    i = pl.program_id(0)

    @pl.when(i < E)
    def _ffn():
        off_e = pl.multiple_of(offs_ref[0, i], ALIGN)
        nch = (cnts_ref[0, i] + TM - 1) // TM

        def body(j, _):
            st = off_e + j * TM
            a = xs_ref[pl.ds(st, TM), :]
            g = jnp.dot(a, w1_ref[0], preferred_element_type=jnp.float32)
            g = g * jax.nn.sigmoid(g)
            v = jnp.dot(a, w3_ref[0], preferred_element_type=jnp.float32)
            ys_ref[pl.ds(st, TM), :] = jnp.dot(
                g * v, w2_ref[0], preferred_element_type=jnp.float32)
            return 0

        lax.fori_loop(0, nch, body, 0)

    @pl.when(i >= E)
    def _combine():
        ti = i - E

        def body(t, _):
            tok = ti * TT + t
            p0 = pos_ref[0, 2 * tok]
            p1 = pos_ref[0, 2 * tok + 1]
            w0 = rwts_ref[0, 2 * tok]
            w1 = rwts_ref[0, 2 * tok + 1]
            y0 = ys_ref[pl.ds(p0, 1), :]
            y1 = ys_ref[pl.ds(p1, 1), :]
            out_ref[pl.ds(t, 1), :] = (shared_ref[pl.ds(t, 1), :]
                                       + w0 * y0 + w1 * y1)
            return 0

        lax.fori_loop(0, TT, body, 0)


def _run_kernel_b(offs, cnts, pos, rwts, xs, rw1, rw2, rw3, shared,
                  interpret=False):
    def wmap(i):
        return (jnp.minimum(i, E - 1), 0, 0)

    def smap(i):
        return (jnp.maximum(i - E, 0), 0)

    _, out = pl.pallas_call(
        _kernel_b,
        grid=(E + NT,),
        in_specs=[
            pl.BlockSpec(memory_space=pltpu.SMEM),
            pl.BlockSpec(memory_space=pltpu.SMEM),
            pl.BlockSpec(memory_space=pltpu.SMEM),
            pl.BlockSpec(memory_space=pltpu.SMEM),
            pl.BlockSpec((XS_ROWS, H), lambda i: (0, 0)),
            pl.BlockSpec((1, H, FFN_R), wmap),
            pl.BlockSpec((1, FFN_R, H), wmap),
            pl.BlockSpec((1, H, FFN_R), wmap),
            pl.BlockSpec((TT, H), smap),
        ],
        out_specs=[
            pl.BlockSpec((XS_ROWS, H), lambda i: (0, 0)),
            pl.BlockSpec((TT, H), smap),
        ],
        out_shape=[
            jax.ShapeDtypeStruct((XS_ROWS, H), jnp.float32),
            jax.ShapeDtypeStruct((T, H), jnp.float32),
        ],
        interpret=interpret,
    )(offs, cnts, pos.reshape(1, NPAIR), rwts.reshape(1, NPAIR), xs,
      rw1, rw2, rw3, shared)
    return out


def kernel(x, shared_norm_w, shared_w1, shared_w2, shared_w3,
           routed_w1, routed_w2, routed_w3, router_down, router_up,
           interpret=False):
    b, t, h = x.shape
    xf = x.reshape(t, h)

    rw, pos, offs, cnts, loss = _run_kernel_a0(
        xf, router_down, router_up, interpret=interpret)

    if interpret:
        xs = _dispatch_jnp(xf, pos)
    else:
        xs = _run_sc_dispatch(xf, pos[:, 0].reshape(-1), pos[:, 1].reshape(-1))

    shared = _run_kernel_a1(xf, shared_norm_w, shared_w1, shared_w2,
                            shared_w3, interpret=interpret)

    out = _run_kernel_b(offs, cnts, pos, rw, xs, routed_w1, routed_w2,
                        routed_w3, shared, interpret=interpret)

    return out.reshape(b, t, h), loss.reshape(())


# speedup vs baseline: 1.0156x; 1.0156x over previous
import functools

import jax
import jax.numpy as jnp
from jax import lax
from jax.experimental import pallas as pl
from jax.experimental.pallas import tpu as pltpu
from jax.experimental.pallas import tpu_sc as plsc

T = 2048
H = 768
E = 64
S_EXP = 2
FFN_S = H * 3
FFN_R = H * 2
R = 64
TOPK = 2
NPAIR = T * TOPK

TT = 256
NT = T // TT
TM = 128
ALIGN = 8
XS_ROWS = 4672


def _kernel_a(x_ref, rd_ref, ru_ref, norm_ref, w1_ref, w2_ref, w3_ref,
              shared_ref, rw_ref, pos_ref, offs_ref, cnts_ref, loss_ref,
              sel_scr, rank_scr, rw_scr, cnt_scr):
    i = pl.program_id(0)
    ids = lax.broadcasted_iota(jnp.int32, (TT, E), 1)

    @pl.when(i == 0)
    def _():
        cnt_scr[...] = jnp.zeros_like(cnt_scr)

    @pl.when(i < NT)
    def _phase0():
        xf = x_ref[...]
        lg = jnp.dot(jnp.dot(xf, rd_ref[...],
                             preferred_element_type=jnp.float32),
                     ru_ref[...], preferred_element_type=jnp.float32)
        v0 = jnp.max(lg, axis=1, keepdims=True)
        e0 = jnp.min(jnp.where(lg == v0, ids, E), axis=1, keepdims=True)
        lg2 = jnp.where(ids == e0, -jnp.inf, lg)
        v1 = jnp.max(lg2, axis=1, keepdims=True)
        e1 = jnp.min(jnp.where(lg2 == v1, ids, E), axis=1, keepdims=True)
        ed = jnp.exp(v1 - v0)
        denom = 1.0 + ed
        sl = pl.ds(i * TT, TT)
        sel_scr[sl, :] = jnp.concatenate([e0, e1], axis=1)
        rw_scr[sl, :] = jnp.concatenate([1.0 / denom, ed / denom], axis=1)

        c0 = (ids == e0).astype(jnp.float32)
        c1 = (ids == e1).astype(jnp.float32)
        m = c0 + c1
        lr = lax.broadcasted_iota(jnp.int32, (TT, TT), 0)
        lc = lax.broadcasted_iota(jnp.int32, (TT, TT), 1)
        ltri = (lr > lc).astype(jnp.float32)
        excl = jnp.dot(ltri, m,
                       preferred_element_type=jnp.float32) + cnt_scr[...]
        rank0 = jnp.sum(excl * c0, axis=1, keepdims=True)
        rank1 = jnp.sum(excl * c1, axis=1, keepdims=True)
        rank_scr[sl, :] = jnp.concatenate([rank0, rank1], axis=1)
        cnt_scr[...] = cnt_scr[...] + jnp.sum(m, axis=0, keepdims=True)

    @pl.when(i >= NT)
    def _phase1():
        xf = x_ref[...]
        inv = lax.rsqrt(jnp.mean(xf * xf, axis=1, keepdims=True) + 1e-6)
        acc = 2.0 * xf
        for s in range(S_EXP):
            hn = xf * inv * norm_ref[s:s + 1, :]
            g = jnp.dot(hn, w1_ref[s], preferred_element_type=jnp.float32)
            g = g * jax.nn.sigmoid(g)
            v = jnp.dot(hn, w3_ref[s], preferred_element_type=jnp.float32)
            acc = acc + jnp.dot(g * v, w2_ref[s],
                                preferred_element_type=jnp.float32)
        shared_ref[...] = acc * (1.0 / S_EXP)

        cnt = cnt_scr[...]
        er = lax.broadcasted_iota(jnp.int32, (E, E), 0)
        ec = lax.broadcasted_iota(jnp.int32, (E, E), 1)
        utri = (er < ec).astype(jnp.float32)
        pc = jnp.ceil(cnt * (1.0 / ALIGN)) * float(ALIGN)
        offs = jnp.dot(pc, utri, preferred_element_type=jnp.float32)
        offs_ref[...] = offs.astype(jnp.int32)
        cnts_ref[...] = cnt.astype(jnp.int32)
        mean = NPAIR / E
        loss_ref[...] = (jnp.sum((cnt - mean) ** 2, keepdims=True)
                         .reshape(1, 1) / (E - 1))

        sl = pl.ds((i - NT) * TT, TT)
        sel = sel_scr[sl, :]
        rank = rank_scr[sl, :]
        on0 = (ids == sel[:, 0:1]).astype(jnp.float32)
        on1 = (ids == sel[:, 1:2]).astype(jnp.float32)
        pos0 = jnp.sum(on0 * offs, axis=1, keepdims=True) + rank[:, 0:1]
        pos1 = jnp.sum(on1 * offs, axis=1, keepdims=True) + rank[:, 1:2]
        pos_ref[...] = jnp.concatenate([pos0, pos1], axis=1).astype(jnp.int32)
        rw_ref[...] = rw_scr[sl, :]


def _run_kernel_a(xf, rd, ru, norm_w, w1, w2, w3, interpret=False):
    def xmap(i):
        return (lax.rem(i, NT), 0)

    def omap(i):
        return (jnp.maximum(i - NT, 0), 0)

    return pl.pallas_call(
        _kernel_a,
        grid=(2 * NT,),
        in_specs=[
            pl.BlockSpec((TT, H), xmap),
            pl.BlockSpec((H, R), lambda i: (0, 0)),
            pl.BlockSpec((R, E), lambda i: (0, 0)),
            pl.BlockSpec((S_EXP, H), lambda i: (0, 0)),
            pl.BlockSpec((S_EXP, H, FFN_S), lambda i: (0, 0, 0)),
            pl.BlockSpec((S_EXP, FFN_S, H), lambda i: (0, 0, 0)),
            pl.BlockSpec((S_EXP, H, FFN_S), lambda i: (0, 0, 0)),
        ],
        out_specs=[
            pl.BlockSpec((TT, H), omap),
            pl.BlockSpec((TT, TOPK), omap),
            pl.BlockSpec((TT, TOPK), omap),
            pl.BlockSpec((1, E), lambda i: (0, 0)),
            pl.BlockSpec((1, E), lambda i: (0, 0)),
            pl.BlockSpec((1, 1), lambda i: (0, 0)),
        ],
        out_shape=[
            jax.ShapeDtypeStruct((T, H), jnp.float32),
            jax.ShapeDtypeStruct((T, TOPK), jnp.float32),
            jax.ShapeDtypeStruct((T, TOPK), jnp.int32),
            jax.ShapeDtypeStruct((1, E), jnp.int32),
            jax.ShapeDtypeStruct((1, E), jnp.int32),
            jax.ShapeDtypeStruct((1, 1), jnp.float32),
        ],
        scratch_shapes=[
            pltpu.VMEM((T, TOPK), jnp.int32),
            pltpu.VMEM((T, TOPK), jnp.float32),
            pltpu.VMEM((T, TOPK), jnp.float32),
            pltpu.VMEM((1, E), jnp.float32),
        ],
        interpret=interpret,
    )(xf, rd, ru, norm_w, w1, w2, w3)


NW = 32
TPW = T // NW


def _sc_dispatch(x_hbm, pos0_hbm, pos1_hbm, xs_hbm, p0_v, p1_v, rows_v,
                 sem0, sem1):
    wid = lax.axis_index("s") * 2 + lax.axis_index("c")
    base = wid * TPW
    pltpu.sync_copy(pos0_hbm.at[pl.ds(base, TPW)], p0_v)
    pltpu.sync_copy(pos1_hbm.at[pl.ds(base, TPW)], p1_v)
    pltpu.sync_copy(x_hbm.at[pl.ds(base, TPW)], rows_v)
    c0 = pltpu.async_copy(rows_v, xs_hbm.at[p0_v], sem0)
    c1 = pltpu.async_copy(rows_v, xs_hbm.at[p1_v], sem1)
    c0.wait()
    c1.wait()


def _run_sc_dispatch(xf, pos0, pos1):
    mesh = plsc.VectorSubcoreMesh(core_axis_name="c", subcore_axis_name="s")
    k = functools.partial(
        pl.kernel, mesh=mesh,
        out_type=jax.ShapeDtypeStruct((XS_ROWS, H), jnp.float32),
        scratch_types=[
            pltpu.VMEM((TPW,), jnp.int32),
            pltpu.VMEM((TPW,), jnp.int32),
            pltpu.VMEM((TPW, H), jnp.float32),
            pltpu.SemaphoreType.DMA,
            pltpu.SemaphoreType.DMA,
        ],
    )(_sc_dispatch)
    return k(xf, pos0, pos1)


def _dispatch_jnp(xf, pos):
    tok = jnp.arange(NPAIR, dtype=jnp.int32) // TOPK
    return jnp.zeros((XS_ROWS, H), jnp.float32).at[pos.reshape(-1)].set(xf[tok])


def _kernel_b(offs_ref, cnts_ref, pos_ref, rwts_ref, xs_ref,
              w1_ref, w2_ref, w3_ref, shared_ref, ys_ref, out_ref):
    i = pl.program_id(0)

    @pl.when(i < E)
    def _ffn():
        off_e = pl.multiple_of(offs_ref[0, i], ALIGN)
        nch = (cnts_ref[0, i] + TM - 1) // TM

        def body(j, _):
            st = off_e + j * TM
            a = xs_ref[pl.ds(st, TM), :]
            g = jnp.dot(a, w1_ref[0], preferred_element_type=jnp.float32)
            g = g * jax.nn.sigmoid(g)
            v = jnp.dot(a, w3_ref[0], preferred_element_type=jnp.float32)
            ys_ref[pl.ds(st, TM), :] = jnp.dot(
                g * v, w2_ref[0], preferred_element_type=jnp.float32)
            return 0

        lax.fori_loop(0, nch, body, 0)

    @pl.when(i >= E)
    def _combine():
        ti = i - E

        def body(t, _):
            tok = ti * TT + t
            p0 = pos_ref[0, 2 * tok]
            p1 = pos_ref[0, 2 * tok + 1]
            w0 = rwts_ref[0, 2 * tok]
            w1 = rwts_ref[0, 2 * tok + 1]
            y0 = ys_ref[pl.ds(p0, 1), :]
            y1 = ys_ref[pl.ds(p1, 1), :]
            out_ref[pl.ds(t, 1), :] = (shared_ref[pl.ds(t, 1), :]
                                       + w0 * y0 + w1 * y1)
            return 0

        lax.fori_loop(0, TT, body, 0)


def _run_kernel_b(offs, cnts, pos, rwts, xs, rw1, rw2, rw3, shared,
                  interpret=False):
    def wmap(i):
        return (jnp.minimum(i, E - 1), 0, 0)

    def smap(i):
        return (jnp.maximum(i - E, 0), 0)

    _, out = pl.pallas_call(
        _kernel_b,
        grid=(E + NT,),
        in_specs=[
            pl.BlockSpec(memory_space=pltpu.SMEM),
            pl.BlockSpec(memory_space=pltpu.SMEM),
            pl.BlockSpec(memory_space=pltpu.SMEM),
            pl.BlockSpec(memory_space=pltpu.SMEM),
            pl.BlockSpec((XS_ROWS, H), lambda i: (0, 0)),
            pl.BlockSpec((1, H, FFN_R), wmap),
            pl.BlockSpec((1, FFN_R, H), wmap),
            pl.BlockSpec((1, H, FFN_R), wmap),
            pl.BlockSpec((TT, H), smap),
        ],
        out_specs=[
            pl.BlockSpec((XS_ROWS, H), lambda i: (0, 0)),
            pl.BlockSpec((TT, H), smap),
        ],
        out_shape=[
            jax.ShapeDtypeStruct((XS_ROWS, H), jnp.float32),
            jax.ShapeDtypeStruct((T, H), jnp.float32),
        ],
        interpret=interpret,
    )(offs, cnts, pos.reshape(1, NPAIR), rwts.reshape(1, NPAIR), xs,
      rw1, rw2, rw3, shared)
    return out


def kernel(x, shared_norm_w, shared_w1, shared_w2, shared_w3,
           routed_w1, routed_w2, routed_w3, router_down, router_up,
           interpret=False):
    b, t, h = x.shape
    xf = x.reshape(t, h)

    shared, rw, pos, offs, cnts, loss = _run_kernel_a(
        xf, router_down, router_up, shared_norm_w, shared_w1, shared_w2,
        shared_w3, interpret=interpret)

    if interpret:
        xs = _dispatch_jnp(xf, pos)
    else:
        xs = _run_sc_dispatch(xf, pos[:, 0].reshape(-1), pos[:, 1].reshape(-1))

    out = _run_kernel_b(offs, cnts, pos, rw, xs, routed_w1, routed_w2,
                        routed_w3, shared, interpret=interpret)

    return out.reshape(b, t, h), loss.reshape(())


# speedup vs baseline: 1.0177x; 1.0021x over previous
import functools

import jax
import jax.numpy as jnp
from jax import lax
from jax.experimental import pallas as pl
from jax.experimental.pallas import tpu as pltpu
from jax.experimental.pallas import tpu_sc as plsc

T = 2048
H = 768
E = 64
S_EXP = 2
FFN_S = H * 3
FFN_R = H * 2
R = 64
TOPK = 2
NPAIR = T * TOPK

TT = 256
NT = T // TT
TM = 128
ALIGN = 8
XS_ROWS = 4672


def _kernel_a(x_ref, rd_ref, ru_ref, norm_ref, w1_ref, w2_ref, w3_ref,
              shared_ref, rw_ref, pos_ref, offs_ref, cnts_ref, loss_ref,
              sel_scr, rank_scr, rw_scr, cnt_scr):
    i = pl.program_id(0)
    ids = lax.broadcasted_iota(jnp.int32, (TT, E), 1)

    @pl.when(i == 0)
    def _():
        cnt_scr[...] = jnp.zeros_like(cnt_scr)

    @pl.when(i < NT)
    def _phase0():
        xf = x_ref[...]
        lg = jnp.dot(jnp.dot(xf, rd_ref[...],
                             preferred_element_type=jnp.float32),
                     ru_ref[...], preferred_element_type=jnp.float32)
        v0 = jnp.max(lg, axis=1, keepdims=True)
        e0 = jnp.min(jnp.where(lg == v0, ids, E), axis=1, keepdims=True)
        lg2 = jnp.where(ids == e0, -jnp.inf, lg)
        v1 = jnp.max(lg2, axis=1, keepdims=True)
        e1 = jnp.min(jnp.where(lg2 == v1, ids, E), axis=1, keepdims=True)
        ed = jnp.exp(v1 - v0)
        denom = 1.0 + ed
        sl = pl.ds(i * TT, TT)
        sel_scr[sl, :] = jnp.concatenate([e0, e1], axis=1)
        rw_scr[sl, :] = jnp.concatenate([1.0 / denom, ed / denom], axis=1)

        c0 = (ids == e0).astype(jnp.float32)
        c1 = (ids == e1).astype(jnp.float32)
        m = c0 + c1
        lr = lax.broadcasted_iota(jnp.int32, (TT, TT), 0)
        lc = lax.broadcasted_iota(jnp.int32, (TT, TT), 1)
        ltri = (lr > lc).astype(jnp.float32)
        excl = jnp.dot(ltri, m,
                       preferred_element_type=jnp.float32) + cnt_scr[...]
        rank0 = jnp.sum(excl * c0, axis=1, keepdims=True)
        rank1 = jnp.sum(excl * c1, axis=1, keepdims=True)
        rank_scr[sl, :] = jnp.concatenate([rank0, rank1], axis=1)
        cnt_scr[...] = cnt_scr[...] + jnp.sum(m, axis=0, keepdims=True)

    @pl.when(i >= NT)
    def _phase1():
        xf = x_ref[...]
        inv = lax.rsqrt(jnp.mean(xf * xf, axis=1, keepdims=True) + 1e-6)
        acc = 2.0 * xf
        for s in range(S_EXP):
            hn = xf * inv * norm_ref[s:s + 1, :]
            g = jnp.dot(hn, w1_ref[s], preferred_element_type=jnp.float32)
            g = g * jax.nn.sigmoid(g)
            v = jnp.dot(hn, w3_ref[s], preferred_element_type=jnp.float32)
            acc = acc + jnp.dot(g * v, w2_ref[s],
                                preferred_element_type=jnp.float32)
        shared_ref[...] = acc * (1.0 / S_EXP)

        cnt = cnt_scr[...]
        er = lax.broadcasted_iota(jnp.int32, (E, E), 0)
        ec = lax.broadcasted_iota(jnp.int32, (E, E), 1)
        utri = (er < ec).astype(jnp.float32)
        pc = jnp.ceil(cnt * (1.0 / ALIGN)) * float(ALIGN)
        offs = jnp.dot(pc, utri, preferred_element_type=jnp.float32)
        offs_ref[...] = offs.astype(jnp.int32)
        cnts_ref[...] = cnt.astype(jnp.int32)
        mean = NPAIR / E
        loss_ref[...] = (jnp.sum((cnt - mean) ** 2, keepdims=True)
                         .reshape(1, 1) / (E - 1))

        sl = pl.ds((i - NT) * TT, TT)
        sel = sel_scr[sl, :]
        rank = rank_scr[sl, :]
        on0 = (ids == sel[:, 0:1]).astype(jnp.float32)
        on1 = (ids == sel[:, 1:2]).astype(jnp.float32)
        pos0 = jnp.sum(on0 * offs, axis=1, keepdims=True) + rank[:, 0:1]
        pos1 = jnp.sum(on1 * offs, axis=1, keepdims=True) + rank[:, 1:2]
        pos_ref[...] = jnp.concatenate([pos0, pos1], axis=1).astype(jnp.int32)
        rw_ref[...] = rw_scr[sl, :]


def _run_kernel_a(xf, rd, ru, norm_w, w1, w2, w3, interpret=False):
    def xmap(i):
        return (lax.rem(i, NT), 0)

    def omap(i):
        return (jnp.maximum(i - NT, 0), 0)

    return pl.pallas_call(
        _kernel_a,
        grid=(2 * NT,),
        in_specs=[
            pl.BlockSpec((TT, H), xmap),
            pl.BlockSpec((H, R), lambda i: (0, 0)),
            pl.BlockSpec((R, E), lambda i: (0, 0)),
            pl.BlockSpec((S_EXP, H), lambda i: (0, 0)),
            pl.BlockSpec((S_EXP, H, FFN_S), lambda i: (0, 0, 0)),
            pl.BlockSpec((S_EXP, FFN_S, H), lambda i: (0, 0, 0)),
            pl.BlockSpec((S_EXP, H, FFN_S), lambda i: (0, 0, 0)),
        ],
        out_specs=[
            pl.BlockSpec((TT, H), omap),
            pl.BlockSpec((TT, TOPK), omap),
            pl.BlockSpec((TT, TOPK), omap),
            pl.BlockSpec((1, E), lambda i: (0, 0)),
            pl.BlockSpec((1, E), lambda i: (0, 0)),
            pl.BlockSpec((1, 1), lambda i: (0, 0)),
        ],
        out_shape=[
            jax.ShapeDtypeStruct((T, H), jnp.float32),
            jax.ShapeDtypeStruct((T, TOPK), jnp.float32),
            jax.ShapeDtypeStruct((T, TOPK), jnp.int32),
            jax.ShapeDtypeStruct((1, E), jnp.int32),
            jax.ShapeDtypeStruct((1, E), jnp.int32),
            jax.ShapeDtypeStruct((1, 1), jnp.float32),
        ],
        scratch_shapes=[
            pltpu.VMEM((T, TOPK), jnp.int32),
            pltpu.VMEM((T, TOPK), jnp.float32),
            pltpu.VMEM((T, TOPK), jnp.float32),
            pltpu.VMEM((1, E), jnp.float32),
        ],
        interpret=interpret,
    )(xf, rd, ru, norm_w, w1, w2, w3)


NW = 32
TPW = T // NW


def _sc_dispatch(x_hbm, pos0_hbm, pos1_hbm, xs_hbm, p0_v, p1_v, rows_v,
                 sem0, sem1, sem2):
    wid = lax.axis_index("s") * 2 + lax.axis_index("c")
    base = wid * TPW
    l0 = pltpu.async_copy(pos0_hbm.at[pl.ds(base, TPW)], p0_v, sem0)
    l1 = pltpu.async_copy(pos1_hbm.at[pl.ds(base, TPW)], p1_v, sem1)
    l2 = pltpu.async_copy(x_hbm.at[pl.ds(base, TPW)], rows_v, sem2)
    l0.wait()
    l1.wait()
    l2.wait()
    c0 = pltpu.async_copy(rows_v, xs_hbm.at[p0_v], sem0)
    c1 = pltpu.async_copy(rows_v, xs_hbm.at[p1_v], sem1)
    c0.wait()
    c1.wait()


def _run_sc_dispatch(xf, pos0, pos1):
    mesh = plsc.VectorSubcoreMesh(core_axis_name="c", subcore_axis_name="s")
    k = functools.partial(
        pl.kernel, mesh=mesh,
        out_type=jax.ShapeDtypeStruct((XS_ROWS, H), jnp.float32),
        scratch_types=[
            pltpu.VMEM((TPW,), jnp.int32),
            pltpu.VMEM((TPW,), jnp.int32),
            pltpu.VMEM((TPW, H), jnp.float32),
            pltpu.SemaphoreType.DMA,
            pltpu.SemaphoreType.DMA,
            pltpu.SemaphoreType.DMA,
        ],
    )(_sc_dispatch)
    return k(xf, pos0, pos1)


def _dispatch_jnp(xf, pos):
    tok = jnp.arange(NPAIR, dtype=jnp.int32) // TOPK
    return jnp.zeros((XS_ROWS, H), jnp.float32).at[pos.reshape(-1)].set(xf[tok])


def _kernel_b(offs_ref, cnts_ref, pos_ref, rwts_ref, xs_ref,
              w1_ref, w2_ref, w3_ref, shared_ref, ys_ref, out_ref):
    i = pl.program_id(0)

    @pl.when(i < E)
    def _ffn():
        off_e = pl.multiple_of(offs_ref[0, i], ALIGN)
        nch = (cnts_ref[0, i] + TM - 1) // TM

        def body(j, _):
            st = off_e + j * TM
            a = xs_ref[pl.ds(st, TM), :]
            g = jnp.dot(a, w1_ref[0], preferred_element_type=jnp.float32)
            g = g * jax.nn.sigmoid(g)
            v = jnp.dot(a, w3_ref[0], preferred_element_type=jnp.float32)
            ys_ref[pl.ds(st, TM), :] = jnp.dot(
                g * v, w2_ref[0], preferred_element_type=jnp.float32)
            return 0

        lax.fori_loop(0, nch, body, 0)

    @pl.when(i >= E)
    def _combine():
        ti = i - E

        def body(t, _):
            tok = ti * TT + t
            p0 = pos_ref[0, 2 * tok]
            p1 = pos_ref[0, 2 * tok + 1]
            w0 = rwts_ref[0, 2 * tok]
            w1 = rwts_ref[0, 2 * tok + 1]
            y0 = ys_ref[pl.ds(p0, 1), :]
            y1 = ys_ref[pl.ds(p1, 1), :]
            out_ref[pl.ds(t, 1), :] = (shared_ref[pl.ds(t, 1), :]
                                       + w0 * y0 + w1 * y1)
            return 0

        lax.fori_loop(0, TT, body, 0)


def _run_kernel_b(offs, cnts, pos, rwts, xs, rw1, rw2, rw3, shared,
                  interpret=False):
    def wmap(i):
        return (jnp.minimum(i, E - 1), 0, 0)

    def smap(i):
        return (jnp.maximum(i - E, 0), 0)

    _, out = pl.pallas_call(
        _kernel_b,
        grid=(E + NT,),
        in_specs=[
            pl.BlockSpec(memory_space=pltpu.SMEM),
            pl.BlockSpec(memory_space=pltpu.SMEM),
            pl.BlockSpec(memory_space=pltpu.SMEM),
            pl.BlockSpec(memory_space=pltpu.SMEM),
            pl.BlockSpec((XS_ROWS, H), lambda i: (0, 0)),
            pl.BlockSpec((1, H, FFN_R), wmap),
            pl.BlockSpec((1, FFN_R, H), wmap),
            pl.BlockSpec((1, H, FFN_R), wmap),
            pl.BlockSpec((TT, H), smap),
        ],
        out_specs=[
            pl.BlockSpec((XS_ROWS, H), lambda i: (0, 0)),
            pl.BlockSpec((TT, H), smap),
        ],
        out_shape=[
            jax.ShapeDtypeStruct((XS_ROWS, H), jnp.float32),
            jax.ShapeDtypeStruct((T, H), jnp.float32),
        ],
        interpret=interpret,
    )(offs, cnts, pos.reshape(1, NPAIR), rwts.reshape(1, NPAIR), xs,
      rw1, rw2, rw3, shared)
    return out


def kernel(x, shared_norm_w, shared_w1, shared_w2, shared_w3,
           routed_w1, routed_w2, routed_w3, router_down, router_up,
           interpret=False):
    b, t, h = x.shape
    xf = x.reshape(t, h)

    shared, rw, pos, offs, cnts, loss = _run_kernel_a(
        xf, router_down, router_up, shared_norm_w, shared_w1, shared_w2,
        shared_w3, interpret=interpret)

    if interpret:
        xs = _dispatch_jnp(xf, pos)
    else:
        xs = _run_sc_dispatch(xf, pos[:, 0].reshape(-1), pos[:, 1].reshape(-1))

    out = _run_kernel_b(offs, cnts, pos, rw, xs, routed_w1, routed_w2,
                        routed_w3, shared, interpret=interpret)

    return out.reshape(b, t, h), loss.reshape(())


# speedup vs baseline: 1.0331x; 1.0151x over previous
import functools

import jax
import jax.numpy as jnp
from jax import lax
from jax.experimental import pallas as pl
from jax.experimental.pallas import tpu as pltpu
from jax.experimental.pallas import tpu_sc as plsc

T = 2048
H = 768
E = 64
S_EXP = 2
FFN_S = H * 3
FFN_R = H * 2
R = 64
TOPK = 2
NPAIR = T * TOPK

TT = 256
NT = T // TT
TM = 128
ALIGN = 8
XS_ROWS = 4672


def _kernel_a(x_ref, rd_ref, ru_ref, norm_ref, w1_ref, w2_ref, w3_ref,
              shared_ref, rw_ref, pos_ref, offs_ref, cnts_ref, loss_ref,
              sel_scr, rank_scr, rw_scr, cnt_scr):
    i = pl.program_id(0)
    ids = lax.broadcasted_iota(jnp.int32, (TT, E), 1)

    @pl.when(i == 0)
    def _():
        cnt_scr[...] = jnp.zeros_like(cnt_scr)

    @pl.when(i < NT)
    def _phase0():
        xf = x_ref[...]
        lg = jnp.dot(jnp.dot(xf, rd_ref[...],
                             preferred_element_type=jnp.float32),
                     ru_ref[...], preferred_element_type=jnp.float32)
        v0 = jnp.max(lg, axis=1, keepdims=True)
        e0 = jnp.min(jnp.where(lg == v0, ids, E), axis=1, keepdims=True)
        lg2 = jnp.where(ids == e0, -jnp.inf, lg)
        v1 = jnp.max(lg2, axis=1, keepdims=True)
        e1 = jnp.min(jnp.where(lg2 == v1, ids, E), axis=1, keepdims=True)
        ed = jnp.exp(v1 - v0)
        denom = 1.0 + ed
        sl = pl.ds(i * TT, TT)
        sel_scr[sl, :] = jnp.concatenate([e0, e1], axis=1)
        rw_scr[sl, :] = jnp.concatenate([1.0 / denom, ed / denom], axis=1)

        c0 = (ids == e0).astype(jnp.float32)
        c1 = (ids == e1).astype(jnp.float32)
        m = c0 + c1
        lr = lax.broadcasted_iota(jnp.int32, (TT, TT), 0)
        lc = lax.broadcasted_iota(jnp.int32, (TT, TT), 1)
        ltri = (lr > lc).astype(jnp.float32)
        excl = jnp.dot(ltri, m,
                       preferred_element_type=jnp.float32) + cnt_scr[...]
        rank0 = jnp.sum(excl * c0, axis=1, keepdims=True)
        rank1 = jnp.sum(excl * c1, axis=1, keepdims=True)
        rank_scr[sl, :] = jnp.concatenate([rank0, rank1], axis=1)
        cnt_scr[...] = cnt_scr[...] + jnp.sum(m, axis=0, keepdims=True)

    @pl.when(i >= NT)
    def _phase1():
        xf = x_ref[...]
        inv = lax.rsqrt(jnp.mean(xf * xf, axis=1, keepdims=True) + 1e-6)
        acc = 2.0 * xf
        for s in range(S_EXP):
            hn = xf * inv * norm_ref[s:s + 1, :]
            g = jnp.dot(hn, w1_ref[s], preferred_element_type=jnp.float32)
            g = g * jax.nn.sigmoid(g)
            v = jnp.dot(hn, w3_ref[s], preferred_element_type=jnp.float32)
            acc = acc + jnp.dot(g * v, w2_ref[s],
                                preferred_element_type=jnp.float32)
        shared_ref[...] = acc * (1.0 / S_EXP)

        cnt = cnt_scr[...]
        er = lax.broadcasted_iota(jnp.int32, (E, E), 0)
        ec = lax.broadcasted_iota(jnp.int32, (E, E), 1)
        utri = (er < ec).astype(jnp.float32)
        pc = jnp.ceil(cnt * (1.0 / ALIGN)) * float(ALIGN)
        offs = jnp.dot(pc, utri, preferred_element_type=jnp.float32)
        offs_ref[...] = offs.astype(jnp.int32)
        cnts_ref[...] = cnt.astype(jnp.int32)
        mean = NPAIR / E
        loss_ref[...] = (jnp.sum((cnt - mean) ** 2, keepdims=True)
                         .reshape(1, 1) / (E - 1))

        sl = pl.ds((i - NT) * TT, TT)
        sel = sel_scr[sl, :]
        rank = rank_scr[sl, :]
        on0 = (ids == sel[:, 0:1]).astype(jnp.float32)
        on1 = (ids == sel[:, 1:2]).astype(jnp.float32)
        pos0 = jnp.sum(on0 * offs, axis=1, keepdims=True) + rank[:, 0:1]
        pos1 = jnp.sum(on1 * offs, axis=1, keepdims=True) + rank[:, 1:2]
        pos_ref[...] = jnp.concatenate([pos0, pos1], axis=1).astype(jnp.int32)
        rw_ref[...] = rw_scr[sl, :]


def _run_kernel_a(xf, rd, ru, norm_w, w1, w2, w3, interpret=False):
    def xmap(i):
        return (lax.rem(i, NT), 0)

    def omap(i):
        return (jnp.maximum(i - NT, 0), 0)

    return pl.pallas_call(
        _kernel_a,
        grid=(2 * NT,),
        in_specs=[
            pl.BlockSpec((TT, H), xmap),
            pl.BlockSpec((H, R), lambda i: (0, 0)),
            pl.BlockSpec((R, E), lambda i: (0, 0)),
            pl.BlockSpec((S_EXP, H), lambda i: (0, 0)),
            pl.BlockSpec((S_EXP, H, FFN_S), lambda i: (0, 0, 0)),
            pl.BlockSpec((S_EXP, FFN_S, H), lambda i: (0, 0, 0)),
            pl.BlockSpec((S_EXP, H, FFN_S), lambda i: (0, 0, 0)),
        ],
        out_specs=[
            pl.BlockSpec((TT, H), omap),
            pl.BlockSpec((TT, TOPK), omap),
            pl.BlockSpec((TT, TOPK), omap),
            pl.BlockSpec((1, E), lambda i: (0, 0)),
            pl.BlockSpec((1, E), lambda i: (0, 0)),
            pl.BlockSpec((1, 1), lambda i: (0, 0)),
        ],
        out_shape=[
            jax.ShapeDtypeStruct((T, H), jnp.float32),
            jax.ShapeDtypeStruct((T, TOPK), jnp.float32),
            jax.ShapeDtypeStruct((T, TOPK), jnp.int32),
            jax.ShapeDtypeStruct((1, E), jnp.int32),
            jax.ShapeDtypeStruct((1, E), jnp.int32),
            jax.ShapeDtypeStruct((1, 1), jnp.float32),
        ],
        scratch_shapes=[
            pltpu.VMEM((T, TOPK), jnp.int32),
            pltpu.VMEM((T, TOPK), jnp.float32),
            pltpu.VMEM((T, TOPK), jnp.float32),
            pltpu.VMEM((1, E), jnp.float32),
        ],
        interpret=interpret,
    )(xf, rd, ru, norm_w, w1, w2, w3)


NW = 32
TPW = T // NW


def _sc_dispatch(x_hbm, pos0_hbm, pos1_hbm, xs_hbm, p0_v, p1_v, rows_v,
                 sem0, sem1, sem2):
    wid = lax.axis_index("s") * 2 + lax.axis_index("c")
    base = wid * TPW
    l0 = pltpu.async_copy(pos0_hbm.at[pl.ds(base, TPW)], p0_v, sem0)
    l1 = pltpu.async_copy(pos1_hbm.at[pl.ds(base, TPW)], p1_v, sem1)
    l2 = pltpu.async_copy(x_hbm.at[pl.ds(base, TPW)], rows_v, sem2)
    l0.wait()
    l1.wait()
    l2.wait()
    c0 = pltpu.async_copy(rows_v, xs_hbm.at[p0_v], sem0)
    c1 = pltpu.async_copy(rows_v, xs_hbm.at[p1_v], sem1)
    c0.wait()
    c1.wait()


def _run_sc_dispatch(xf, pos0, pos1):
    mesh = plsc.VectorSubcoreMesh(core_axis_name="c", subcore_axis_name="s")
    k = functools.partial(
        pl.kernel, mesh=mesh,
        out_type=jax.ShapeDtypeStruct((XS_ROWS, H), jnp.float32),
        scratch_types=[
            pltpu.VMEM((TPW,), jnp.int32),
            pltpu.VMEM((TPW,), jnp.int32),
            pltpu.VMEM((TPW, H), jnp.float32),
            pltpu.SemaphoreType.DMA,
            pltpu.SemaphoreType.DMA,
            pltpu.SemaphoreType.DMA,
        ],
    )(_sc_dispatch)
    return k(xf, pos0, pos1)


def _dispatch_jnp(xf, pos):
    tok = jnp.arange(NPAIR, dtype=jnp.int32) // TOPK
    return jnp.zeros((XS_ROWS, H), jnp.float32).at[pos.reshape(-1)].set(xf[tok])


def _kernel_b(offs_ref, cnts_ref, pos_ref, rwts_ref, xs_ref,
              w1_ref, w2_ref, w3_ref, shared_ref, ys_ref, out_ref):
    i = pl.program_id(0)

    @pl.when(i < E)
    def _ffn():
        off_e = pl.multiple_of(offs_ref[0, i], ALIGN)
        nch = (cnts_ref[0, i] + TM - 1) // TM

        def body(j, _):
            st = off_e + j * TM
            a = xs_ref[pl.ds(st, TM), :]
            g = jnp.dot(a, w1_ref[0], preferred_element_type=jnp.float32)
            g = g * jax.nn.sigmoid(g)
            v = jnp.dot(a, w3_ref[0], preferred_element_type=jnp.float32)
            ys_ref[pl.ds(st, TM), :] = jnp.dot(
                g * v, w2_ref[0], preferred_element_type=jnp.float32)
            return 0

        lax.fori_loop(0, nch, body, 0)

    @pl.when(i >= E)
    def _combine():
        ti = i - E

        def body(t8, _):
            rows = []
            for k in range(8):
                t = t8 * 8 + k
                tok = ti * TT + t
                p0 = pos_ref[0, 2 * tok]
                p1 = pos_ref[0, 2 * tok + 1]
                w0 = rwts_ref[0, 2 * tok]
                w1 = rwts_ref[0, 2 * tok + 1]
                rows.append(shared_ref[pl.ds(t, 1), :]
                            + w0 * ys_ref[pl.ds(p0, 1), :]
                            + w1 * ys_ref[pl.ds(p1, 1), :])
            out_ref[pl.ds(t8 * 8, 8), :] = jnp.concatenate(rows, axis=0)
            return 0

        lax.fori_loop(0, TT // 8, body, 0)


def _run_kernel_b(offs, cnts, pos, rwts, xs, rw1, rw2, rw3, shared,
                  interpret=False):
    def wmap(i):
        return (jnp.minimum(i, E - 1), 0, 0)

    def smap(i):
        return (jnp.maximum(i - E, 0), 0)

    _, out = pl.pallas_call(
        _kernel_b,
        grid=(E + NT,),
        in_specs=[
            pl.BlockSpec(memory_space=pltpu.SMEM),
            pl.BlockSpec(memory_space=pltpu.SMEM),
            pl.BlockSpec(memory_space=pltpu.SMEM),
            pl.BlockSpec(memory_space=pltpu.SMEM),
            pl.BlockSpec((XS_ROWS, H), lambda i: (0, 0)),
            pl.BlockSpec((1, H, FFN_R), wmap),
            pl.BlockSpec((1, FFN_R, H), wmap),
            pl.BlockSpec((1, H, FFN_R), wmap),
            pl.BlockSpec((TT, H), smap),
        ],
        out_specs=[
            pl.BlockSpec((XS_ROWS, H), lambda i: (0, 0)),
            pl.BlockSpec((TT, H), smap),
        ],
        out_shape=[
            jax.ShapeDtypeStruct((XS_ROWS, H), jnp.float32),
            jax.ShapeDtypeStruct((T, H), jnp.float32),
        ],
        interpret=interpret,
    )(offs, cnts, pos.reshape(1, NPAIR), rwts.reshape(1, NPAIR), xs,
      rw1, rw2, rw3, shared)
    return out


def kernel(x, shared_norm_w, shared_w1, shared_w2, shared_w3,
           routed_w1, routed_w2, routed_w3, router_down, router_up,
           interpret=False):
    b, t, h = x.shape
    xf = x.reshape(t, h)

    shared, rw, pos, offs, cnts, loss = _run_kernel_a(
        xf, router_down, router_up, shared_norm_w, shared_w1, shared_w2,
        shared_w3, interpret=interpret)

    if interpret:
        xs = _dispatch_jnp(xf, pos)
    else:
        xs = _run_sc_dispatch(xf, pos[:, 0].reshape(-1), pos[:, 1].reshape(-1))

    out = _run_kernel_b(offs, cnts, pos, rw, xs, routed_w1, routed_w2,
                        routed_w3, shared, interpret=interpret)

    return out.reshape(b, t, h), loss.reshape(())


# speedup vs baseline: 1.0337x; 1.0006x over previous
import functools

import jax
import jax.numpy as jnp
from jax import lax
from jax.experimental import pallas as pl
from jax.experimental.pallas import tpu as pltpu
from jax.experimental.pallas import tpu_sc as plsc

T = 2048
H = 768
E = 64
S_EXP = 2
FFN_S = H * 3
FFN_R = H * 2
R = 64
TOPK = 2
NPAIR = T * TOPK

TT = 256
NT = T // TT
TM = 128
ALIGN = 8
XS_ROWS = 4672


def _kernel_a(x_ref, rd_ref, ru_ref, norm_ref, w1_ref, w2_ref, w3_ref,
              shared_ref, rw_ref, pos_ref, offs_ref, cnts_ref, loss_ref,
              sel_scr, rank_scr, rw_scr, cnt_scr):
    i = pl.program_id(0)
    ids = lax.broadcasted_iota(jnp.int32, (TT, E), 1)

    @pl.when(i == 0)
    def _():
        cnt_scr[...] = jnp.zeros_like(cnt_scr)

    @pl.when(i < NT)
    def _phase0():
        xf = x_ref[...]
        lg = jnp.dot(jnp.dot(xf, rd_ref[...],
                             preferred_element_type=jnp.float32),
                     ru_ref[...], preferred_element_type=jnp.float32)
        v0 = jnp.max(lg, axis=1, keepdims=True)
        e0 = jnp.min(jnp.where(lg == v0, ids, E), axis=1, keepdims=True)
        lg2 = jnp.where(ids == e0, -jnp.inf, lg)
        v1 = jnp.max(lg2, axis=1, keepdims=True)
        e1 = jnp.min(jnp.where(lg2 == v1, ids, E), axis=1, keepdims=True)
        ed = jnp.exp(v1 - v0)
        denom = 1.0 + ed
        sl = pl.ds(i * TT, TT)
        sel_scr[sl, :] = jnp.concatenate([e0, e1], axis=1)
        rw_scr[sl, :] = jnp.concatenate([1.0 / denom, ed / denom], axis=1)

        c0 = (ids == e0).astype(jnp.float32)
        c1 = (ids == e1).astype(jnp.float32)
        m = c0 + c1
        lr = lax.broadcasted_iota(jnp.int32, (TT, TT), 0)
        lc = lax.broadcasted_iota(jnp.int32, (TT, TT), 1)
        ltri = (lr > lc).astype(jnp.float32)
        excl = jnp.dot(ltri, m,
                       preferred_element_type=jnp.float32) + cnt_scr[...]
        rank0 = jnp.sum(excl * c0, axis=1, keepdims=True)
        rank1 = jnp.sum(excl * c1, axis=1, keepdims=True)
        rank_scr[sl, :] = jnp.concatenate([rank0, rank1], axis=1)
        cnt_scr[...] = cnt_scr[...] + jnp.sum(m, axis=0, keepdims=True)

    @pl.when(i >= NT)
    def _phase1():
        xf = x_ref[...]
        inv = lax.rsqrt(jnp.mean(xf * xf, axis=1, keepdims=True) + 1e-6)
        acc = 2.0 * xf
        for s in range(S_EXP):
            hn = xf * inv * norm_ref[s:s + 1, :]
            g = jnp.dot(hn, w1_ref[s], preferred_element_type=jnp.float32)
            g = g * jax.nn.sigmoid(g)
            v = jnp.dot(hn, w3_ref[s], preferred_element_type=jnp.float32)
            acc = acc + jnp.dot(g * v, w2_ref[s],
                                preferred_element_type=jnp.float32)
        shared_ref[...] = acc * (1.0 / S_EXP)

        cnt = cnt_scr[...]
        er = lax.broadcasted_iota(jnp.int32, (E, E), 0)
        ec = lax.broadcasted_iota(jnp.int32, (E, E), 1)
        utri = (er < ec).astype(jnp.float32)
        pc = jnp.ceil(cnt * (1.0 / ALIGN)) * float(ALIGN)
        offs = jnp.dot(pc, utri, preferred_element_type=jnp.float32)
        offs_ref[...] = offs.astype(jnp.int32)
        cnts_ref[...] = cnt.astype(jnp.int32)
        mean = NPAIR / E
        loss_ref[...] = (jnp.sum((cnt - mean) ** 2, keepdims=True)
                         .reshape(1, 1) / (E - 1))

        sl = pl.ds((i - NT) * TT, TT)
        sel = sel_scr[sl, :]
        rank = rank_scr[sl, :]
        on0 = (ids == sel[:, 0:1]).astype(jnp.float32)
        on1 = (ids == sel[:, 1:2]).astype(jnp.float32)
        pos0 = jnp.sum(on0 * offs, axis=1, keepdims=True) + rank[:, 0:1]
        pos1 = jnp.sum(on1 * offs, axis=1, keepdims=True) + rank[:, 1:2]
        pos_ref[...] = jnp.concatenate([pos0, pos1], axis=1).astype(jnp.int32)
        rw_ref[...] = rw_scr[sl, :]


def _run_kernel_a(xf, rd, ru, norm_w, w1, w2, w3):
    def xmap(i):
        return (lax.rem(i, NT), 0)

    def omap(i):
        return (jnp.maximum(i - NT, 0), 0)

    return pl.pallas_call(
        _kernel_a,
        grid=(2 * NT,),
        in_specs=[
            pl.BlockSpec((TT, H), xmap),
            pl.BlockSpec((H, R), lambda i: (0, 0)),
            pl.BlockSpec((R, E), lambda i: (0, 0)),
            pl.BlockSpec((S_EXP, H), lambda i: (0, 0)),
            pl.BlockSpec((S_EXP, H, FFN_S), lambda i: (0, 0, 0)),
            pl.BlockSpec((S_EXP, FFN_S, H), lambda i: (0, 0, 0)),
            pl.BlockSpec((S_EXP, H, FFN_S), lambda i: (0, 0, 0)),
        ],
        out_specs=[
            pl.BlockSpec((TT, H), omap),
            pl.BlockSpec((TT, TOPK), omap),
            pl.BlockSpec((TT, TOPK), omap),
            pl.BlockSpec((1, E), lambda i: (0, 0)),
            pl.BlockSpec((1, E), lambda i: (0, 0)),
            pl.BlockSpec((1, 1), lambda i: (0, 0)),
        ],
        out_shape=[
            jax.ShapeDtypeStruct((T, H), jnp.float32),
            jax.ShapeDtypeStruct((T, TOPK), jnp.float32),
            jax.ShapeDtypeStruct((T, TOPK), jnp.int32),
            jax.ShapeDtypeStruct((1, E), jnp.int32),
            jax.ShapeDtypeStruct((1, E), jnp.int32),
            jax.ShapeDtypeStruct((1, 1), jnp.float32),
        ],
        scratch_shapes=[
            pltpu.VMEM((T, TOPK), jnp.int32),
            pltpu.VMEM((T, TOPK), jnp.float32),
            pltpu.VMEM((T, TOPK), jnp.float32),
            pltpu.VMEM((1, E), jnp.float32),
        ],
    )(xf, rd, ru, norm_w, w1, w2, w3)


NW = 32
TPW = T // NW


def _sc_dispatch(x_hbm, pos0_hbm, pos1_hbm, xs_hbm, p0_v, p1_v, rows_v,
                 sem0, sem1, sem2):
    wid = lax.axis_index("s") * 2 + lax.axis_index("c")
    base = wid * TPW
    l0 = pltpu.async_copy(pos0_hbm.at[pl.ds(base, TPW)], p0_v, sem0)
    l1 = pltpu.async_copy(pos1_hbm.at[pl.ds(base, TPW)], p1_v, sem1)
    l2 = pltpu.async_copy(x_hbm.at[pl.ds(base, TPW)], rows_v, sem2)
    l0.wait()
    l1.wait()
    l2.wait()
    c0 = pltpu.async_copy(rows_v, xs_hbm.at[p0_v], sem0)
    c1 = pltpu.async_copy(rows_v, xs_hbm.at[p1_v], sem1)
    c0.wait()
    c1.wait()


def _run_sc_dispatch(xf, pos0, pos1):
    mesh = plsc.VectorSubcoreMesh(core_axis_name="c", subcore_axis_name="s")
    k = functools.partial(
        pl.kernel, mesh=mesh,
        out_type=jax.ShapeDtypeStruct((XS_ROWS, H), jnp.float32),
        scratch_types=[
            pltpu.VMEM((TPW,), jnp.int32),
            pltpu.VMEM((TPW,), jnp.int32),
            pltpu.VMEM((TPW, H), jnp.float32),
            pltpu.SemaphoreType.DMA,
            pltpu.SemaphoreType.DMA,
            pltpu.SemaphoreType.DMA,
        ],
    )(_sc_dispatch)
    return k(xf, pos0, pos1)


def _kernel_b(offs_ref, cnts_ref, pos_ref, rwts_ref, xs_ref,
              w1_ref, w2_ref, w3_ref, shared_ref, ys_ref, out_ref):
    i = pl.program_id(0)

    @pl.when(i < E)
    def _ffn():
        off_e = pl.multiple_of(offs_ref[0, i], ALIGN)
        nch = (cnts_ref[0, i] + TM - 1) // TM

        def body(j, _):
            st = off_e + j * TM
            a = xs_ref[pl.ds(st, TM), :]
            g = jnp.dot(a, w1_ref[0], preferred_element_type=jnp.float32)
            g = g * jax.nn.sigmoid(g)
            v = jnp.dot(a, w3_ref[0], preferred_element_type=jnp.float32)
            ys_ref[pl.ds(st, TM), :] = jnp.dot(
                g * v, w2_ref[0], preferred_element_type=jnp.float32)
            return 0

        lax.fori_loop(0, nch, body, 0)

    @pl.when(i >= E)
    def _combine():
        ti = i - E

        def body(t8, _):
            rows = []
            for k in range(8):
                t = t8 * 8 + k
                tok = ti * TT + t
                p0 = pos_ref[0, 2 * tok]
                p1 = pos_ref[0, 2 * tok + 1]
                w0 = rwts_ref[0, 2 * tok]
                w1 = rwts_ref[0, 2 * tok + 1]
                rows.append(shared_ref[pl.ds(t, 1), :]
                            + w0 * ys_ref[pl.ds(p0, 1), :]
                            + w1 * ys_ref[pl.ds(p1, 1), :])
            out_ref[pl.ds(t8 * 8, 8), :] = jnp.concatenate(rows, axis=0)
            return 0

        lax.fori_loop(0, TT // 8, body, 0)


def _run_kernel_b(offs, cnts, pos, rwts, xs, rw1, rw2, rw3, shared):
    def wmap(i):
        return (jnp.minimum(i, E - 1), 0, 0)

    def smap(i):
        return (jnp.maximum(i - E, 0), 0)

    _, out = pl.pallas_call(
        _kernel_b,
        grid=(E + NT,),
        in_specs=[
            pl.BlockSpec(memory_space=pltpu.SMEM),
            pl.BlockSpec(memory_space=pltpu.SMEM),
            pl.BlockSpec(memory_space=pltpu.SMEM),
            pl.BlockSpec(memory_space=pltpu.SMEM),
            pl.BlockSpec((XS_ROWS, H), lambda i: (0, 0)),
            pl.BlockSpec((1, H, FFN_R), wmap),
            pl.BlockSpec((1, FFN_R, H), wmap),
            pl.BlockSpec((1, H, FFN_R), wmap),
            pl.BlockSpec((TT, H), smap),
        ],
        out_specs=[
            pl.BlockSpec((XS_ROWS, H), lambda i: (0, 0)),
            pl.BlockSpec((TT, H), smap),
        ],
        out_shape=[
            jax.ShapeDtypeStruct((XS_ROWS, H), jnp.float32),
            jax.ShapeDtypeStruct((T, H), jnp.float32),
        ],
    )(offs, cnts, pos.reshape(1, NPAIR), rwts.reshape(1, NPAIR), xs,
      rw1, rw2, rw3, shared)
    return out


def kernel(x, shared_norm_w, shared_w1, shared_w2, shared_w3,
           routed_w1, routed_w2, routed_w3, router_down, router_up):
    b, t, h = x.shape
    xf = x.reshape(t, h)

    shared, rw, pos, offs, cnts, loss = _run_kernel_a(
        xf, router_down, router_up, shared_norm_w, shared_w1, shared_w2,
        shared_w3)

    xs = _run_sc_dispatch(xf, pos[:, 0].reshape(-1), pos[:, 1].reshape(-1))

    out = _run_kernel_b(offs, cnts, pos, rw, xs, routed_w1, routed_w2,
                        routed_w3, shared)

    return out.reshape(b, t, h), loss.reshape(())
